# baseline (speedup 1.0000x reference)
import functools
import jax
import jax.numpy as jnp
from jax import lax
from jax.experimental import pallas as pl
from jax.experimental.pallas import tpu as pltpu
from jax.experimental.pallas import tpu_sc as plsc

B = 16384
K = 16
NU = 1000000
NM = 100000
NC = 2
NS = 16
L = 16
NW = NC * NS
BPW = B // NW
NG = BPW // L

BL = 131072


def _detile_body(x_ref, o_ref):
    x = x_ref[...]
    o_ref[...] = (
        x.reshape(2, 8, BL // 128, 128)
        .transpose(0, 2, 1, 3)
        .reshape(BL * K)
    )


def _detile_call(xt, n):
    nblk = (n + BL - 1) // BL
    return pl.pallas_call(
        _detile_body,
        grid=(nblk,),
        in_specs=[pl.BlockSpec((K, BL), lambda i: (0, i))],
        out_specs=pl.BlockSpec((BL * K,), lambda i: (i,)),
        out_shape=jax.ShapeDtypeStruct((nblk * BL * K,), jnp.float32),
    )(xt)


_mesh = plsc.VectorSubcoreMesh(core_axis_name="c", subcore_axis_name="s")


@functools.partial(
    pl.kernel,
    mesh=_mesh,
    out_type=jax.ShapeDtypeStruct((B,), jnp.float32),
    compiler_params=pltpu.CompilerParams(
        needs_layout_passes=False, use_tc_tiling_on_sc=False
    ),
    scratch_types=[
        pltpu.VMEM((BPW,), jnp.int32),
        pltpu.VMEM((BPW,), jnp.int32),
        pltpu.VMEM((K * BPW,), jnp.int32),
        pltpu.VMEM((K * BPW,), jnp.int32),
        pltpu.VMEM((K * BPW,), jnp.float32),
        pltpu.VMEM((K * BPW,), jnp.float32),
        pltpu.VMEM((BPW,), jnp.float32),
        pltpu.VMEM((BPW,), jnp.float32),
        pltpu.VMEM((BPW,), jnp.float32),
        pltpu.SemaphoreType.DMA,
    ],
)
def _mf_kernel(users_hbm, items_hbm, uf_hbm, mf_hbm, ub_hbm, mb_hbm, out_hbm,
               uidx_v, iidx_v, uoff_v, moff_v, uval_v, mval_v, ub_v, mb_v,
               out_v, sem):
    wid = lax.axis_index("s") * NC + lax.axis_index("c")
    base = wid * BPW

    pltpu.sync_copy(users_hbm.at[pl.ds(base, BPW)], uidx_v)
    pltpu.sync_copy(items_hbm.at[pl.ds(base, BPW)], iidx_v)

    cp_ub = pltpu.async_copy(ub_hbm.at[uidx_v], ub_v, sem)
    cp_mb = pltpu.async_copy(mb_hbm.at[iidx_v], mb_v, sem)

    def offsets(g, _):
        b0 = g * L
        rv = uidx_v[pl.ds(b0, L)]
        iv = iidx_v[pl.ds(b0, L)]
        ubase = ((rv // BL) * (BL * K) + ((rv // 128) % (BL // 128)) * 1024
                 + rv % 128)
        mbase = ((iv // BL) * (BL * K) + ((iv // 128) % (BL // 128)) * 1024
                 + iv % 128)
        for k in range(K):
            dk = (k // 8) * (BL * 8) + (k % 8) * 128
            uoff_v[pl.ds(k * BPW + b0, L)] = ubase + dk
            moff_v[pl.ds(k * BPW + b0, L)] = mbase + dk
        return 0

    lax.fori_loop(0, NG, offsets, 0)

    cp_u = pltpu.async_copy(uf_hbm.at[uoff_v], uval_v, sem)
    cp_m = pltpu.async_copy(mf_hbm.at[moff_v], mval_v, sem)
    cp_u.wait()
    cp_m.wait()
    cp_ub.wait()
    cp_mb.wait()

    def group(g, _):
        b0 = g * L
        acc = ub_v[pl.ds(b0, L)] + mb_v[pl.ds(b0, L)]
        for k in range(K):
            uk = uval_v[pl.ds(k * BPW + b0, L)]
            mk = mval_v[pl.ds(k * BPW + b0, L)]
            acc = acc + uk * mk
        out_v[pl.ds(b0, L)] = acc
        return 0

    lax.fori_loop(0, NG, group, 0)

    pltpu.sync_copy(out_v, out_hbm.at[pl.ds(base, BPW)])


def kernel(users, items, U, M, Ub, Mb):
    uflat = _detile_call(U.T, NU)
    mflat = _detile_call(M.T, NM)
    return _mf_kernel(users, items, uflat, mflat,
                      Ub.reshape(-1), Mb.reshape(-1))


# speedup vs baseline: 1.0080x; 1.0080x over previous
import functools
import jax
import jax.numpy as jnp
from jax import lax
from jax.experimental import pallas as pl
from jax.experimental.pallas import tpu as pltpu
from jax.experimental.pallas import tpu_sc as plsc

B = 16384
K = 16
NU = 1000000
NM = 100000
NC = 2
NS = 16
L = 16
NW = NC * NS
BPW = B // NW
NG = BPW // L

BL = 65536


def _detile_body(x_ref, o_ref):
    x = x_ref[...]
    o_ref[...] = (
        x.reshape(2, 8, BL // 128, 128)
        .transpose(0, 2, 1, 3)
        .reshape(BL * K)
    )


def _detile_call(xt, n):
    nblk = (n + BL - 1) // BL
    return pl.pallas_call(
        _detile_body,
        grid=(nblk,),
        in_specs=[pl.BlockSpec((K, BL), lambda i: (0, i))],
        out_specs=pl.BlockSpec((BL * K,), lambda i: (i,)),
        out_shape=jax.ShapeDtypeStruct((nblk * BL * K,), jnp.float32),
    )(xt)


_mesh = plsc.VectorSubcoreMesh(core_axis_name="c", subcore_axis_name="s")


@functools.partial(
    pl.kernel,
    mesh=_mesh,
    out_type=jax.ShapeDtypeStruct((B,), jnp.float32),
    compiler_params=pltpu.CompilerParams(
        needs_layout_passes=False, use_tc_tiling_on_sc=False
    ),
    scratch_types=[
        pltpu.VMEM((BPW,), jnp.int32),
        pltpu.VMEM((BPW,), jnp.int32),
        pltpu.VMEM((K * BPW,), jnp.int32),
        pltpu.VMEM((K * BPW,), jnp.int32),
        pltpu.VMEM((K * BPW,), jnp.float32),
        pltpu.VMEM((K * BPW,), jnp.float32),
        pltpu.VMEM((BPW,), jnp.float32),
        pltpu.VMEM((BPW,), jnp.float32),
        pltpu.VMEM((BPW,), jnp.float32),
        pltpu.SemaphoreType.DMA,
    ],
)
def _mf_kernel(users_hbm, items_hbm, uf_hbm, mf_hbm, ub_hbm, mb_hbm, out_hbm,
               uidx_v, iidx_v, uoff_v, moff_v, uval_v, mval_v, ub_v, mb_v,
               out_v, sem):
    wid = lax.axis_index("s") * NC + lax.axis_index("c")
    base = wid * BPW

    pltpu.sync_copy(users_hbm.at[pl.ds(base, BPW)], uidx_v)
    pltpu.sync_copy(items_hbm.at[pl.ds(base, BPW)], iidx_v)

    cp_ub = pltpu.async_copy(ub_hbm.at[uidx_v], ub_v, sem)
    cp_mb = pltpu.async_copy(mb_hbm.at[iidx_v], mb_v, sem)

    def offsets(g, _):
        b0 = g * L
        rv = uidx_v[pl.ds(b0, L)]
        iv = iidx_v[pl.ds(b0, L)]
        ubase = ((rv // BL) * (BL * K) + ((rv // 128) % (BL // 128)) * 1024
                 + rv % 128)
        mbase = ((iv // BL) * (BL * K) + ((iv // 128) % (BL // 128)) * 1024
                 + iv % 128)
        for k in range(K):
            dk = (k // 8) * (BL * 8) + (k % 8) * 128
            uoff_v[pl.ds(k * BPW + b0, L)] = ubase + dk
            moff_v[pl.ds(k * BPW + b0, L)] = mbase + dk
        return 0

    lax.fori_loop(0, NG, offsets, 0)

    cp_u = pltpu.async_copy(uf_hbm.at[uoff_v], uval_v, sem)
    cp_m = pltpu.async_copy(mf_hbm.at[moff_v], mval_v, sem)
    cp_u.wait()
    cp_m.wait()
    cp_ub.wait()
    cp_mb.wait()

    def group(g, _):
        b0 = g * L
        acc = ub_v[pl.ds(b0, L)] + mb_v[pl.ds(b0, L)]
        for k in range(K):
            uk = uval_v[pl.ds(k * BPW + b0, L)]
            mk = mval_v[pl.ds(k * BPW + b0, L)]
            acc = acc + uk * mk
        out_v[pl.ds(b0, L)] = acc
        return 0

    lax.fori_loop(0, NG, group, 0)

    pltpu.sync_copy(out_v, out_hbm.at[pl.ds(base, BPW)])


def kernel(users, items, U, M, Ub, Mb):
    uflat = _detile_call(U.T, NU)
    mflat = _detile_call(M.T, NM)
    return _mf_kernel(users, items, uflat, mflat,
                      Ub.reshape(-1), Mb.reshape(-1))


# speedup vs baseline: 1.0092x; 1.0012x over previous
import functools
import jax
import jax.numpy as jnp
from jax import lax
from jax.experimental import pallas as pl
from jax.experimental.pallas import tpu as pltpu
from jax.experimental.pallas import tpu_sc as plsc

B = 16384
K = 16
NU = 1000000
NM = 100000
NC = 2
NS = 16
L = 16
NW = NC * NS
BPW = B // NW
NG = BPW // L

BL = 65536


def _detile_body(x_ref, o_ref):
    x = x_ref[...]
    o_ref[...] = (
        x.reshape(2, 8, BL // 128, 128)
        .transpose(0, 2, 1, 3)
        .reshape(BL * K)
    )


def _detile_call(xt, n):
    nblk = (n + BL - 1) // BL
    return pl.pallas_call(
        _detile_body,
        grid=(nblk,),
        in_specs=[pl.BlockSpec((K, BL), lambda i: (0, i))],
        out_specs=pl.BlockSpec((BL * K,), lambda i: (i,)),
        out_shape=jax.ShapeDtypeStruct((nblk * BL * K,), jnp.float32),
    )(xt)


_mesh = plsc.VectorSubcoreMesh(core_axis_name="c", subcore_axis_name="s")


@functools.partial(
    pl.kernel,
    mesh=_mesh,
    out_type=jax.ShapeDtypeStruct((B,), jnp.float32),
    compiler_params=pltpu.CompilerParams(
        needs_layout_passes=False, use_tc_tiling_on_sc=False
    ),
    scratch_types=[
        pltpu.VMEM((BPW,), jnp.int32),
        pltpu.VMEM((BPW,), jnp.int32),
        pltpu.VMEM((K * BPW,), jnp.int32),
        pltpu.VMEM((K * BPW,), jnp.int32),
        pltpu.VMEM((K * BPW,), jnp.float32),
        pltpu.VMEM((K * BPW,), jnp.float32),
        pltpu.VMEM((BPW,), jnp.float32),
        pltpu.VMEM((BPW,), jnp.float32),
        pltpu.VMEM((BPW,), jnp.float32),
        pltpu.SemaphoreType.DMA,
    ],
)
def _mf_kernel(users_hbm, items_hbm, uf_hbm, mf_hbm, ub_hbm, mb_hbm, out_hbm,
               uidx_v, iidx_v, uoff_v, moff_v, uval_v, mval_v, ub_v, mb_v,
               out_v, sem):
    wid = lax.axis_index("s") * NC + lax.axis_index("c")
    base = wid * BPW

    pltpu.sync_copy(users_hbm.at[pl.ds(base, BPW)], uidx_v)
    pltpu.sync_copy(items_hbm.at[pl.ds(base, BPW)], iidx_v)

    cp_ub = pltpu.async_copy(ub_hbm.at[uidx_v], ub_v, sem)
    cp_mb = pltpu.async_copy(mb_hbm.at[iidx_v], mb_v, sem)

    def offsets(g, _):
        b0 = g * L
        rv = uidx_v[pl.ds(b0, L)]
        iv = iidx_v[pl.ds(b0, L)]
        ubase = ((rv // BL) * (BL * K) + ((rv // 128) % (BL // 128)) * 1024
                 + rv % 128)
        mbase = ((iv // BL) * (BL * K) + ((iv // 128) % (BL // 128)) * 1024
                 + iv % 128)
        for k in range(K):
            dk = (k // 8) * (BL * 8) + (k % 8) * 128
            uoff_v[pl.ds(k * BPW + b0, L)] = ubase + dk
            moff_v[pl.ds(k * BPW + b0, L)] = mbase + dk
        return 0

    lax.fori_loop(0, NG, offsets, 0)

    cp_u = pltpu.async_copy(uf_hbm.at[uoff_v], uval_v, sem)
    cp_m = pltpu.async_copy(mf_hbm.at[moff_v], mval_v, sem)
    cp_u.wait()
    cp_m.wait()
    cp_ub.wait()
    cp_mb.wait()

    def group(g, _):
        b0 = g * L
        acc = ub_v[pl.ds(b0, L)] + mb_v[pl.ds(b0, L)]
        for k in range(K):
            uk = uval_v[pl.ds(k * BPW + b0, L)]
            mk = mval_v[pl.ds(k * BPW + b0, L)]
            acc = acc + uk * mk
        out_v[pl.ds(b0, L)] = acc
        return 0

    lax.fori_loop(0, NG, group, 0)

    pltpu.sync_copy(out_v, out_hbm.at[pl.ds(base, BPW)])


def kernel(users, items, U, M, Ub, Mb):
    uflat = _detile_call(U.T, NU)
    mflat = _detile_call(M.T, NM)
    return _mf_kernel(users, items, uflat, mflat,
                      Ub.T.reshape(-1), Mb.T.reshape(-1))


# speedup vs baseline: 1.3530x; 1.3406x over previous
import functools
import jax
import jax.numpy as jnp
from jax import lax
from jax.experimental import pallas as pl
from jax.experimental.pallas import tpu as pltpu
from jax.experimental.pallas import tpu_sc as plsc

B = 16384
K = 16
NU = 1000000
NM = 100000
NC = 2
NS = 16
L = 16
NW = NC * NS
BPW = B // NW
NG = BPW // L

BL = 65536


def _detile_body(x_ref, o_ref):
    x = x_ref[...]
    o_ref[...] = (
        x.reshape(2, 8, BL // 128, 128)
        .transpose(0, 2, 1, 3)
        .reshape(BL * K)
    )


def _flatten_bias_body(x_ref, o_ref):
    o_ref[...] = x_ref[...].reshape(BL)


def _flatten_bias(bt, n):
    nblk = (n + BL - 1) // BL
    return pl.pallas_call(
        _flatten_bias_body,
        grid=(nblk,),
        in_specs=[pl.BlockSpec((1, BL), lambda i: (0, i))],
        out_specs=pl.BlockSpec((BL,), lambda i: (i,)),
        out_shape=jax.ShapeDtypeStruct((nblk * BL,), jnp.float32),
    )(bt)


def _detile_call(xt, n):
    nblk = (n + BL - 1) // BL
    return pl.pallas_call(
        _detile_body,
        grid=(nblk,),
        in_specs=[pl.BlockSpec((K, BL), lambda i: (0, i))],
        out_specs=pl.BlockSpec((BL * K,), lambda i: (i,)),
        out_shape=jax.ShapeDtypeStruct((nblk * BL * K,), jnp.float32),
    )(xt)


_mesh = plsc.VectorSubcoreMesh(core_axis_name="c", subcore_axis_name="s")


@functools.partial(
    pl.kernel,
    mesh=_mesh,
    out_type=jax.ShapeDtypeStruct((B,), jnp.float32),
    compiler_params=pltpu.CompilerParams(
        needs_layout_passes=False, use_tc_tiling_on_sc=False
    ),
    scratch_types=[
        pltpu.VMEM((BPW,), jnp.int32),
        pltpu.VMEM((BPW,), jnp.int32),
        pltpu.VMEM((K * BPW,), jnp.int32),
        pltpu.VMEM((K * BPW,), jnp.int32),
        pltpu.VMEM((K * BPW,), jnp.float32),
        pltpu.VMEM((K * BPW,), jnp.float32),
        pltpu.VMEM((BPW,), jnp.float32),
        pltpu.VMEM((BPW,), jnp.float32),
        pltpu.VMEM((BPW,), jnp.float32),
        pltpu.SemaphoreType.DMA,
    ],
)
def _mf_kernel(users_hbm, items_hbm, uf_hbm, mf_hbm, ub_hbm, mb_hbm, out_hbm,
               uidx_v, iidx_v, uoff_v, moff_v, uval_v, mval_v, ub_v, mb_v,
               out_v, sem):
    wid = lax.axis_index("s") * NC + lax.axis_index("c")
    base = wid * BPW

    pltpu.sync_copy(users_hbm.at[pl.ds(base, BPW)], uidx_v)
    pltpu.sync_copy(items_hbm.at[pl.ds(base, BPW)], iidx_v)

    cp_ub = pltpu.async_copy(ub_hbm.at[uidx_v], ub_v, sem)
    cp_mb = pltpu.async_copy(mb_hbm.at[iidx_v], mb_v, sem)

    def offsets(g, _):
        b0 = g * L
        rv = uidx_v[pl.ds(b0, L)]
        iv = iidx_v[pl.ds(b0, L)]
        ubase = ((rv // BL) * (BL * K) + ((rv // 128) % (BL // 128)) * 1024
                 + rv % 128)
        mbase = ((iv // BL) * (BL * K) + ((iv // 128) % (BL // 128)) * 1024
                 + iv % 128)
        for k in range(K):
            dk = (k // 8) * (BL * 8) + (k % 8) * 128
            uoff_v[pl.ds(k * BPW + b0, L)] = ubase + dk
            moff_v[pl.ds(k * BPW + b0, L)] = mbase + dk
        return 0

    lax.fori_loop(0, NG, offsets, 0)

    cp_u = pltpu.async_copy(uf_hbm.at[uoff_v], uval_v, sem)
    cp_m = pltpu.async_copy(mf_hbm.at[moff_v], mval_v, sem)
    cp_u.wait()
    cp_m.wait()
    cp_ub.wait()
    cp_mb.wait()

    def group(g, _):
        b0 = g * L
        acc = ub_v[pl.ds(b0, L)] + mb_v[pl.ds(b0, L)]
        for k in range(K):
            uk = uval_v[pl.ds(k * BPW + b0, L)]
            mk = mval_v[pl.ds(k * BPW + b0, L)]
            acc = acc + uk * mk
        out_v[pl.ds(b0, L)] = acc
        return 0

    lax.fori_loop(0, NG, group, 0)

    pltpu.sync_copy(out_v, out_hbm.at[pl.ds(base, BPW)])


def kernel(users, items, U, M, Ub, Mb):
    uflat = _detile_call(U.T, NU)
    mflat = _detile_call(M.T, NM)
    return _mf_kernel(users, items, uflat, mflat,
                      _flatten_bias(Ub.T, NU), _flatten_bias(Mb.T, NM))
